# baseline (speedup 1.0000x reference)
import functools

import jax
import jax.numpy as jnp
from jax import lax
from jax.experimental import pallas as pl
from jax.experimental.pallas import tpu as pltpu
from jax.experimental.pallas import tpu_sc as plsc

B = 16384
D = 64
K = 20
VOC = 1000000
NC = 2
NS = 16
NW = NC * NS
NSLOT = B * (K + 1)
NTILE = 7813
TBASE = NTILE // NW
TEXTRA = NTILE - TBASE * NW

SR1 = 1024
NS1 = 31
CAP1 = 784
SCAP1 = 112

SR2 = 512
NB2 = 8
CAP2 = 12784
CAPB = 1776
SCAP2 = 448
CAPW = 21504

EROWS = 16400
SENT = 0x7FFFFFFF
VOC_AL = (VOC // 128) * 128

_mesh = plsc.VectorSubcoreMesh(
    core_axis_name="c", subcore_axis_name="s", num_cores=NC, num_subcores=NS)
_cparams = pltpu.CompilerParams(
    needs_layout_passes=False, use_tc_tiling_on_sc=True)


def _worker_range(wid):
    t0 = wid * TBASE + jnp.minimum(wid, TEXTRA)
    nt = TBASE + jnp.where(wid < TEXTRA, 1, 0)
    wlo = t0 * 128
    whi = jnp.minimum((t0 + nt) * 128, VOC)
    return wlo, whi


def _u_body(pos_u_hbm, ut_hbm, utail_hbm, emb_hbm,
            pu, staged, uh_v, uh_e, sh_c, sh_e, scat2d, rows_out, sem):
    wid = lax.axis_index("s") * NC + lax.axis_index("c")
    wlo, whi = _worker_range(wid)
    iota = lax.iota(jnp.int32, 16)

    pltpu.sync_copy(pos_u_hbm, pu)

    def scan_body(g, nh):
        v = pu[pl.ds(g * 16, 16)]
        m = jnp.logical_and(v >= wlo, v < whi)
        base = jnp.minimum(nh, CAP1 - 16)
        plsc.store_compressed(uh_v.at[pl.ds(base, 16)], v, mask=m)
        plsc.store_compressed(uh_e.at[pl.ds(base, 16)], g * 16 + iota, mask=m)
        return jnp.minimum(nh + plsc.all_reduce_population_count(m)[0], CAP1 - 16)

    nh = lax.fori_loop(0, B // 16, scan_body, jnp.int32(0))

    def sr_body(s, carry):
        lo = wlo + s * SR1
        hi = jnp.minimum(lo + SR1, whi)

        @pl.when(lo < whi)
        def _():
            is_tail = hi > VOC_AL
            c0 = pl.multiple_of(jnp.minimum(lo, VOC_AL - SR1), 128)
            sbase = jnp.where(is_tail, VOC - SR1, c0)

            @pl.when(jnp.logical_not(is_tail))
            def _():
                pltpu.async_copy(
                    ut_hbm.at[pl.ds(0, D), pl.ds(c0, SR1)], staged, sem).wait()

            @pl.when(is_tail)
            def _():
                pltpu.async_copy(utail_hbm, staged, sem).wait()

            def pre_body(t, c2):
                sh_e[pl.ds(t * 16, 16)] = jnp.full((16,), EROWS - 16, jnp.int32)
                return c2
            lax.fori_loop(0, (SCAP1 + 32) // 16, pre_body, 0)

            def sscan_body(g, ns):
                v = uh_v[pl.ds(g * 16, 16)]
                e = uh_e[pl.ds(g * 16, 16)]
                m = jnp.logical_and(v >= lo, v < hi)
                base = jnp.minimum(ns, SCAP1)
                plsc.store_compressed(sh_c.at[pl.ds(base, 16)], v - sbase,
                                      mask=m)
                plsc.store_compressed(sh_e.at[pl.ds(base, 16)], e, mask=m)
                return jnp.minimum(
                    ns + plsc.all_reduce_population_count(m)[0], SCAP1)

            ns = lax.fori_loop(0, (nh + 15) // 16, sscan_body, jnp.int32(0))
            for jj in range(8):
                scat2d[jj] = sh_e[pl.ds(jj * 16, 16)]

            def ext_body(h, c2):
                c = sh_c[pl.ds(h, 16)][0]
                cvec = jnp.full((16,), c, jnp.int32)
                for q in range(4):
                    rows_out[h, pl.ds(q * 16, 16)] = plsc.load_gather(
                        staged, [q * 16 + iota, cvec])
                return c2

            lax.fori_loop(0, ns, ext_body, 0)

            def scat_body(j, c2):
                pltpu.async_copy(
                    rows_out.at[pl.ds(j * 16, 16)],
                    emb_hbm.at[scat2d.at[j]], sem).wait()
                return c2

            lax.fori_loop(0, (ns + 15) // 16, scat_body, 0)
        return carry

    lax.fori_loop(0, NS1, sr_body, 0)


_sc_uphase = pl.kernel(
    _u_body,
    out_type=jax.ShapeDtypeStruct((EROWS, 2 * D), jnp.float32),
    mesh=_mesh,
    compiler_params=_cparams,
    scratch_types=[
        pltpu.VMEM((B,), jnp.int32),
        pltpu.VMEM((D, SR1), jnp.float32),
        pltpu.VMEM((CAP1 + 16,), jnp.int32),
        pltpu.VMEM((CAP1 + 16,), jnp.int32),
        pltpu.VMEM((SCAP1 + 32,), jnp.int32),
        pltpu.VMEM((SCAP1 + 32,), jnp.int32),
        pltpu.VMEM((8, 16), jnp.int32),
        pltpu.VMEM((SCAP1 + 16, 2 * D), jnp.float32),
        pltpu.SemaphoreType.DMA,
    ],
)


def _v_body(vall_hbm, ue2_hbm, vt_hbm, vtail_hbm, emb_hbm, sc_pk_hbm, id_pk_hbm,
            vbuf, uebuf, wh, bk, nbbuf, sh_loc, sh_ue, staged, urows, ebuf,
            cumbuf, sc_stage, neg1, sem):
    wid = lax.axis_index("s") * NC + lax.axis_index("c")
    wlo, whi = _worker_range(wid)
    iota = lax.iota(jnp.int32, 16)
    rowstart = iota * 16

    def pre_wh(g, c):
        wh[pl.ds(g * 16, 16)] = jnp.full((16,), SENT, jnp.int32)
        return c
    lax.fori_loop(0, (CAP2 + 16) // 16, pre_wh, 0)

    def pre_bk(g, c):
        bk[pl.ds(g * 16, 16)] = jnp.full((16,), SENT, jnp.int32)
        return c
    lax.fori_loop(0, NB2 * (CAPB + 16) // 16, pre_bk, 0)
    for t in range(8):
        neg1[pl.ds(t * 16, 16)] = jnp.full((16,), -1, jnp.int32)

    def chunk_body(cc, nh):
        pltpu.sync_copy(vall_hbm.at[pl.ds(cc * 2048, 2048)], vbuf)
        pltpu.sync_copy(ue2_hbm.at[pl.ds(cc * 2048, 2048)], uebuf)

        def scan_body(g, nh2):
            v = vbuf[pl.ds(g * 16, 16)]
            u = uebuf[pl.ds(g * 16, 16)]
            m = jnp.logical_and(v >= wlo, v < whi)
            h = jnp.bitwise_or(lax.shift_left(v - wlo, 15), u)
            base = jnp.minimum(nh2, CAP2 - 16)
            plsc.store_compressed(wh.at[pl.ds(base, 16)], h, mask=m)
            return jnp.minimum(
                nh2 + plsc.all_reduce_population_count(m)[0], CAP2 - 16)

        return lax.fori_loop(0, 128, scan_body, nh)

    nh = lax.fori_loop(0, NSLOT // 2048, chunk_body, jnp.int32(0))

    def place_body(g, counts):
        h = wh[pl.ds(g * 16, 16)]
        b = lax.shift_right_logical(h, 27)
        new = []
        for k in range(NB2):
            m = b == k
            base = jnp.minimum(counts[k], CAPB)
            plsc.store_compressed(
                bk.at[pl.ds(k * (CAPB + 16) + base, 16)], h, mask=m)
            new.append(jnp.minimum(
                counts[k] + plsc.all_reduce_population_count(m)[0], CAPB))
        return tuple(new)

    counts = lax.fori_loop(0, (nh + 15) // 16, place_body,
                           tuple(jnp.int32(0) for _ in range(NB2)))
    nbv = jnp.zeros((16,), jnp.int32)
    for k in range(NB2):
        nbv = jnp.where(iota == k, counts[k], nbv)
    nbbuf[pl.ds(0, 16)] = nbv

    def bucket_body(k, off):
        nbk = plsc.load_gather(nbbuf, [jnp.full((16,), k, jnp.int32)])[0]

        def sr_body(si, off2):
            s = k * 8 + si
            lo = wlo + s * SR2
            hi = jnp.minimum(lo + SR2, whi)

            def do_block(off3):
                is_tail = hi > VOC_AL
                c0 = pl.multiple_of(jnp.minimum(lo, VOC_AL - SR2), 128)
                sbase = jnp.where(is_tail, VOC - SR2, c0)

                @pl.when(jnp.logical_not(is_tail))
                def _():
                    pltpu.async_copy(
                        vt_hbm.at[pl.ds(0, D), pl.ds(c0, SR2)],
                        staged, sem).wait()

                @pl.when(is_tail)
                def _():
                    pltpu.async_copy(vtail_hbm, staged, sem).wait()

                def pre_sh(t, c2):
                    sh_ue[pl.ds(t * 16, 16)] = jnp.full((16,), -1, jnp.int32)
                    return c2
                lax.fori_loop(0, (SCAP2 + 64) // 16, pre_sh, 0)

                lo_l = lo - wlo
                hi_l = hi - wlo
                dc = sbase - wlo

                def rescan_body(g, ns):
                    h = bk[pl.ds(k * (CAPB + 16) + g * 16, 16)]
                    loc = lax.shift_right_logical(h, 15)
                    m = jnp.logical_and(loc >= lo_l, loc < hi_l)
                    base = jnp.minimum(ns, SCAP2)
                    plsc.store_compressed(
                        sh_loc.at[pl.ds(base, 16)], loc - dc, mask=m)
                    plsc.store_compressed(
                        sh_ue.at[pl.ds(base, 16)],
                        jnp.bitwise_and(h, 32767), mask=m)
                    return jnp.minimum(
                        ns + plsc.all_reduce_population_count(m)[0], SCAP2)

                ns = lax.fori_loop(0, (nbk + 15) // 16, rescan_body,
                                   jnp.int32(0))

                def wave_body(j, off4):
                    for t in range(8):
                        uev = sh_ue[pl.ds(j * 128 + t * 16, 16)]
                        ebuf[pl.ds(t * 16, 16)] = jnp.where(
                            uev < 0, EROWS - 16,
                            lax.shift_right_logical(uev, 1))
                    pltpu.async_copy(emb_hbm.at[ebuf], urows, sem).wait()

                    def hit_body(h2, c2):
                        cl = sh_loc[pl.ds(j * 128 + h2, 16)][0]
                        cvec = jnp.full((16,), cl, jnp.int32)
                        acc = plsc.load_gather(staged, [iota, cvec]) \
                            * urows[h2, pl.ds(0, 16)]
                        for q in range(1, 4):
                            acc = acc + plsc.load_gather(
                                staged, [q * 16 + iota, cvec]) \
                                * urows[h2, pl.ds(q * 16, 16)]
                        cumbuf[pl.ds(h2 * 16, 16)] = acc
                        return c2

                    lax.fori_loop(0, jnp.minimum(128, ns - j * 128),
                                  hit_body, 0)

                    def fin_body(g, c2):
                        t2 = plsc.load_gather(cumbuf, [g * 256 + rowstart])
                        for jj in range(1, 16):
                            t2 = t2 + plsc.load_gather(
                                cumbuf, [g * 256 + rowstart + jj])
                        sc_stage[pl.ds(g * 16, 16)] = t2
                        return c2

                    lax.fori_loop(0, 8, fin_body, 0)
                    obase = pl.multiple_of(wid * CAPW + off4, 128)
                    pltpu.sync_copy(sc_stage, sc_pk_hbm.at[pl.ds(obase, 128)])
                    pltpu.sync_copy(sh_ue.at[pl.ds(j * 128, 128)],
                                    id_pk_hbm.at[pl.ds(obase, 128)])
                    return off4 + 128

                return lax.fori_loop(0, (ns + 127) // 128, wave_body, off3)

            return lax.cond(lo < whi, do_block, lambda o: o, off2)

        return lax.fori_loop(0, 8, sr_body, off)

    off = lax.fori_loop(0, NB2, bucket_body, jnp.int32(0))

    def fill_body(t, c):
        fbase = pl.multiple_of(wid * CAPW + off + t * 128, 128)
        pltpu.sync_copy(neg1, id_pk_hbm.at[pl.ds(fbase, 128)])
        return c

    lax.fori_loop(0, (CAPW - off) // 128, fill_body, 0)


_sc_vphase = pl.kernel(
    _v_body,
    out_type=(jax.ShapeDtypeStruct((NW * CAPW,), jnp.float32),
              jax.ShapeDtypeStruct((NW * CAPW,), jnp.int32)),
    mesh=_mesh,
    compiler_params=_cparams,
    scratch_types=[
        pltpu.VMEM((2048,), jnp.int32),
        pltpu.VMEM((2048,), jnp.int32),
        pltpu.VMEM((CAP2 + 16,), jnp.int32),
        pltpu.VMEM((NB2 * (CAPB + 16),), jnp.int32),
        pltpu.VMEM((16,), jnp.int32),
        pltpu.VMEM((SCAP2 + 64,), jnp.int32),
        pltpu.VMEM((SCAP2 + 64,), jnp.int32),
        pltpu.VMEM((D, SR2), jnp.float32),
        pltpu.VMEM((128, 2 * D), jnp.float32),
        pltpu.VMEM((128,), jnp.int32),
        pltpu.VMEM((2048,), jnp.float32),
        pltpu.VMEM((128,), jnp.float32),
        pltpu.VMEM((128,), jnp.int32),
        pltpu.SemaphoreType.DMA,
    ],
)

_TC_ROWS = NW * CAPW // 128


def _tc_body(x_ref, id_ref, o_ref):
    x = x_ref[...]
    i = id_ref[...]
    valid = i >= 0
    is_pos = jnp.logical_and(valid, jnp.bitwise_and(i, 1) == 0)
    x = jnp.where(valid, x, 0.0)
    xc = jnp.clip(x, -10.0, 10.0)
    t = jnp.where(is_pos, -xc, xc)
    term = jnp.log1p(jnp.exp(t))
    term = jnp.where(valid, term, 0.0)
    pos_mean = jnp.sum(jnp.where(is_pos, term, 0.0)) * (1.0 / B)
    neg_mean = jnp.sum(jnp.where(is_pos, 0.0, term)) * (1.0 / (B * K))
    lane = lax.broadcasted_iota(jnp.int32, (1, 128), 1)
    o_ref[...] = jnp.where(lane == 0, pos_mean,
                           jnp.where(lane == 1, neg_mean, 0.0))


_tc_loss = pl.pallas_call(
    _tc_body,
    out_shape=jax.ShapeDtypeStruct((1, 128), jnp.float32),
)


def kernel(pos_u, pos_v, neg_v, u_table, v_table):
    ut = u_table.T
    vt = v_table.T
    vall = jnp.concatenate([pos_v, neg_v.reshape(-1)])
    elem = jnp.arange(B, dtype=jnp.int32)
    ue2 = jnp.concatenate(
        [elem * 2, (jnp.repeat(elem, K) * 2 + 1)]).astype(jnp.int32)
    utail = lax.slice(ut, (0, VOC - SR1), (D, VOC))
    vtail = lax.slice(vt, (0, VOC - SR2), (D, VOC))
    emb = _sc_uphase(pos_u, ut, utail)
    sc_pk, id_pk = _sc_vphase(vall, ue2, vt, vtail, emb)
    sums = _tc_loss(sc_pk.reshape(_TC_ROWS, 128), id_pk.reshape(_TC_ROWS, 128))
    a = sums[0, 0]
    b = sums[0, 1]
    return (a + b, a, b)
